# Initial kernel scaffold; baseline (speedup 1.0000x reference)
import functools
import math

import jax
import jax.numpy as jnp
from jax import lax
from jax.experimental import pallas as pl
from jax.experimental.pallas import tpu as pltpu
from jax.experimental.pallas import tpu_sc as plsc

VOCAB = 100000
D = 64
BATCH = 1024
SEQ = 200
THETA = 10000.0

NC, NS, L = 2, 16, 16
NW = NC * NS
ROWS = BATCH * SEQ
RPW = ROWS // NW
CHUNK = 100
NCHUNK = RPW // CHUNK


def _rope_coeffs():
    scale = math.sqrt(D)
    freqs = 1.0 / (THETA ** (jnp.arange(0, D, 2).astype(jnp.float32) / D))
    pos = jnp.arange(SEQ).astype(jnp.float32)[:, None]
    cos = jnp.cos(pos * freqs[None, :]) * scale
    sin = jnp.sin(pos * freqs[None, :]) * scale
    a = jnp.repeat(cos, 2, axis=1)
    b = jnp.stack([-sin, sin], axis=-1).reshape(SEQ, D)
    return a, b


_GATHER_DNUMS = lax.GatherDimensionNumbers(
    offset_dims=(), collapsed_slice_dims=(0,), start_index_map=(0,))


def _swap_pairs(v, perm2d):
    return lax.gather(v, perm2d, dimension_numbers=_GATHER_DNUMS,
                      slice_sizes=(1,),
                      mode=lax.GatherScatterMode.PROMISE_IN_BOUNDS)


def _sc_body(table_hbm, x_hbm, a_hbm, b_hbm, out_hbm,
             idx_v, in0, in1, out0, out1, a_v, b_v,
             gsem0, gsem1, ssem0, ssem1):
    w = lax.axis_index("s") * NC + lax.axis_index("c")
    row_base = w * RPW

    pltpu.sync_copy(x_hbm.at[w], idx_v)
    pltpu.sync_copy(a_hbm, a_v)
    pltpu.sync_copy(b_hbm, b_v)

    perm2d = (lax.iota(jnp.int32, 16) ^ 1)[:, None]

    inbufs = (in0, in1)
    outbufs = (out0, out1)
    gsems = (gsem0, gsem1)
    ssems = (ssem0, ssem1)

    def start_gather(c, j):
        pltpu.async_copy(table_hbm.at[idx_v.at[c]], inbufs[j], gsems[j])

    def wait_gather(j):
        pltpu.make_async_copy(table_hbm.at[pl.ds(0, CHUNK)],
                              inbufs[j], gsems[j]).wait()

    def start_store(c, j):
        pltpu.async_copy(outbufs[j],
                         out_hbm.at[pl.ds(row_base + c * CHUNK, CHUNK)],
                         ssems[j])

    def wait_store(j):
        pltpu.make_async_copy(outbufs[j],
                              out_hbm.at[pl.ds(row_base, CHUNK)],
                              ssems[j]).wait()

    def compute(j):
        pbase = j * CHUNK
        inb = inbufs[j]
        outb = outbufs[j]

        def row(i, _):
            p = pbase + i
            for k in range(D // L):
                sl = pl.ds(k * L, L)
                v = inb[i, sl]
                sw = _swap_pairs(v, perm2d)
                outb[i, sl] = v * a_v[p, sl] + sw * b_v[p, sl]
            return 0

        lax.fori_loop(0, CHUNK, row, 0)

    start_gather(0, 0)

    def step(t, _):
        for j in range(2):
            c = 2 * t + j

            @pl.when(c < NCHUNK - 1)
            def _():
                start_gather(c + 1, 1 - j)

            wait_gather(j)

            @pl.when(c >= 2)
            def _():
                wait_store(j)

            compute(j)
            start_store(c, j)
        return 0

    lax.fori_loop(0, NCHUNK // 2, step, 0)
    wait_store(0)
    wait_store(1)


@jax.jit
def _run(table, x_resh, a, b):
    mesh = plsc.VectorSubcoreMesh(core_axis_name="c", subcore_axis_name="s")
    kern = pl.kernel(
        _sc_body,
        out_type=jax.ShapeDtypeStruct((ROWS, D), jnp.float32),
        mesh=mesh,
        scratch_types=[
            pltpu.VMEM((NCHUNK, CHUNK), jnp.int32),
            pltpu.VMEM((CHUNK, D), jnp.float32),
            pltpu.VMEM((CHUNK, D), jnp.float32),
            pltpu.VMEM((CHUNK, D), jnp.float32),
            pltpu.VMEM((CHUNK, D), jnp.float32),
            pltpu.VMEM((SEQ, D), jnp.float32),
            pltpu.VMEM((SEQ, D), jnp.float32),
            pltpu.SemaphoreType.DMA,
            pltpu.SemaphoreType.DMA,
            pltpu.SemaphoreType.DMA,
            pltpu.SemaphoreType.DMA,
        ],
        name="rope_embed_sc",
    )
    return kern(table, x_resh, a, b)


def kernel(x, table):
    a, b = _rope_coeffs()
    x_resh = x.astype(jnp.int32).reshape(NW, NCHUNK, CHUNK)
    out = _run(table, x_resh, a, b)
    return out.reshape(BATCH, SEQ, D)


# speedup vs baseline: 2.8608x; 2.8608x over previous
import functools
import math

import jax
import jax.numpy as jnp
from jax import lax
from jax.experimental import pallas as pl
from jax.experimental.pallas import tpu as pltpu
from jax.experimental.pallas import tpu_sc as plsc

VOCAB = 100000
D = 64
BATCH = 1024
SEQ = 200
THETA = 10000.0

NC, NS, L = 2, 16, 16
NW = NC * NS
ROWS = BATCH * SEQ
RPW = ROWS // NW
CHUNK = 128
NCHUNK = RPW // CHUNK


def _rope_coeffs():
    scale = math.sqrt(D)
    freqs = 1.0 / (THETA ** (jnp.arange(0, D, 2).astype(jnp.float32) / D))
    pos = jnp.arange(SEQ).astype(jnp.float32)[:, None]
    cos = jnp.cos(pos * freqs[None, :]) * scale
    sin = jnp.sin(pos * freqs[None, :]) * scale
    a = jnp.repeat(cos, 2, axis=1)
    b = jnp.stack([-sin, sin], axis=-1).reshape(SEQ, D)
    return a, b


_GATHER_DNUMS = lax.GatherDimensionNumbers(
    offset_dims=(), collapsed_slice_dims=(0,), start_index_map=(0,))


def _swap_pairs(v, perm2d):
    return lax.gather(v, perm2d, dimension_numbers=_GATHER_DNUMS,
                      slice_sizes=(1,),
                      mode=lax.GatherScatterMode.PROMISE_IN_BOUNDS)


def _sc_body(table_hbm, x_hbm, a_hbm, b_hbm, out_hbm,
             idx_v, in0, in1, out0, out1, a_v, b_v,
             gsem0, gsem1, ssem0, ssem1):
    w = lax.axis_index("s") * NC + lax.axis_index("c")
    row_base = w * RPW

    pltpu.sync_copy(x_hbm.at[w], idx_v)
    pltpu.sync_copy(a_hbm, a_v)
    pltpu.sync_copy(b_hbm, b_v)

    perm2d = (lax.iota(jnp.int32, 16) ^ 1)[:, None]

    inbufs = (in0, in1)
    outbufs = (out0, out1)
    gsems = (gsem0, gsem1)
    ssems = (ssem0, ssem1)

    def start_gather(c, j):
        pltpu.async_copy(table_hbm.at[idx_v.at[c]], inbufs[j], gsems[j])

    def wait_gather(j):
        pltpu.make_async_copy(table_hbm.at[pl.ds(0, CHUNK)],
                              inbufs[j], gsems[j]).wait()

    def start_store(c, j):
        pltpu.async_copy(outbufs[j],
                         out_hbm.at[pl.ds(row_base + c * CHUNK, CHUNK)],
                         ssems[j])

    def wait_store(j):
        pltpu.make_async_copy(outbufs[j],
                              out_hbm.at[pl.ds(row_base, CHUNK)],
                              ssems[j]).wait()

    def compute(c, j):
        pbase = c * CHUNK
        inb = inbufs[j]
        outb = outbufs[j]

        def row(i, _):
            p = lax.rem(pbase + i, SEQ)
            for k in range(D // L):
                sl = pl.ds(k * L, L)
                v = inb[i, sl]
                sw = _swap_pairs(v, perm2d)
                outb[i, sl] = v * a_v[p, sl] + sw * b_v[p, sl]
            return 0

        lax.fori_loop(0, CHUNK, row, 0)

    start_gather(0, 0)

    def step(t, _):
        for j in range(2):
            c = 2 * t + j

            @pl.when(c < NCHUNK - 1)
            def _():
                start_gather(c + 1, 1 - j)

            wait_gather(j)

            @pl.when(c >= 2)
            def _():
                wait_store(j)

            compute(c, j)
            start_store(c, j)
        return 0

    lax.fori_loop(0, NCHUNK // 2, step, 0)
    wait_store(0)
    wait_store(1)


@jax.jit
def _run(table, x_resh, a, b):
    mesh = plsc.VectorSubcoreMesh(core_axis_name="c", subcore_axis_name="s")
    kern = pl.kernel(
        _sc_body,
        out_type=jax.ShapeDtypeStruct((ROWS, D), jnp.float32),
        mesh=mesh,
        scratch_types=[
            pltpu.VMEM((NCHUNK, CHUNK), jnp.int32),
            pltpu.VMEM((CHUNK, D), jnp.float32),
            pltpu.VMEM((CHUNK, D), jnp.float32),
            pltpu.VMEM((CHUNK, D), jnp.float32),
            pltpu.VMEM((CHUNK, D), jnp.float32),
            pltpu.VMEM((SEQ, D), jnp.float32),
            pltpu.VMEM((SEQ, D), jnp.float32),
            pltpu.SemaphoreType.DMA,
            pltpu.SemaphoreType.DMA,
            pltpu.SemaphoreType.DMA,
            pltpu.SemaphoreType.DMA,
        ],
        compiler_params=pltpu.CompilerParams(use_tc_tiling_on_sc=False),
        name="rope_embed_sc",
    )
    return kern(table, x_resh, a, b)


def kernel(x, table):
    a, b = _rope_coeffs()
    x_resh = x.astype(jnp.int32).reshape(NW, NCHUNK, CHUNK)
    out = _run(table, x_resh, a, b)
    return out.reshape(BATCH, SEQ, D)


# speedup vs baseline: 4.0717x; 1.4233x over previous
import functools
import math

import jax
import jax.numpy as jnp
from jax import lax
from jax.experimental import pallas as pl
from jax.experimental.pallas import tpu as pltpu
from jax.experimental.pallas import tpu_sc as plsc

VOCAB = 100000
D = 64
BATCH = 1024
SEQ = 200
THETA = 10000.0

NC, NS, L = 2, 16, 16
NW = NC * NS
ROWS = BATCH * SEQ
RPW = ROWS // NW
CHUNK = 128
NCHUNK = RPW // CHUNK


def _rope_coeffs():
    scale = math.sqrt(D)
    freqs = 1.0 / (THETA ** (jnp.arange(0, D, 2).astype(jnp.float32) / D))
    pos = jnp.arange(SEQ).astype(jnp.float32)[:, None]
    cos = jnp.cos(pos * freqs[None, :]) * scale
    sin = jnp.sin(pos * freqs[None, :]) * scale
    a = jnp.repeat(cos, 2, axis=1)
    b = jnp.stack([-sin, sin], axis=-1).reshape(SEQ, D)
    return a, b


_GATHER_DNUMS = lax.GatherDimensionNumbers(
    offset_dims=(), collapsed_slice_dims=(0,), start_index_map=(0,))


def _swap_pairs(v, perm2d):
    return lax.gather(v, perm2d, dimension_numbers=_GATHER_DNUMS,
                      slice_sizes=(1,),
                      mode=lax.GatherScatterMode.PROMISE_IN_BOUNDS)


def _sc_body(table_hbm, x_hbm, a_hbm, b_hbm, out_hbm,
             idx_v, in0, in1, out0, out1, a_v, b_v,
             gsem0, gsem1, ssem0, ssem1):
    w = lax.axis_index("s") * NC + lax.axis_index("c")
    row_base = w * RPW

    pltpu.sync_copy(x_hbm.at[w], idx_v)
    pltpu.sync_copy(a_hbm, a_v)
    pltpu.sync_copy(b_hbm, b_v)

    perm2d = (lax.iota(jnp.int32, 16) ^ 1)[:, None]

    inbufs = (in0, in1)
    outbufs = (out0, out1)
    gsems = (gsem0, gsem1)
    ssems = (ssem0, ssem1)

    def start_gather(c, j):
        pltpu.async_copy(table_hbm.at[idx_v.at[c]], inbufs[j], gsems[j])

    def wait_gather(j):
        pltpu.make_async_copy(table_hbm.at[pl.ds(0, CHUNK)],
                              inbufs[j], gsems[j]).wait()

    def start_store(c, j):
        pltpu.async_copy(outbufs[j],
                         out_hbm.at[pl.ds(row_base + c * CHUNK, CHUNK)],
                         ssems[j])

    def wait_store(j):
        pltpu.make_async_copy(outbufs[j],
                              out_hbm.at[pl.ds(row_base, CHUNK)],
                              ssems[j]).wait()

    def compute(c, j):
        pbase = c * CHUNK
        inb = inbufs[j]
        outb = outbufs[j]

        @plsc.parallel_loop(0, CHUNK, 1, unroll=8)
        def row(i):
            p = lax.rem(pbase + i, SEQ)
            for k in range(D // L):
                sl = pl.ds(k * L, L)
                v = inb[i, sl]
                sw = _swap_pairs(v, perm2d)
                outb[i, sl] = v * a_v[p, sl] + sw * b_v[p, sl]

    start_gather(0, 0)

    def step(t, _):
        for j in range(2):
            c = 2 * t + j

            @pl.when(c < NCHUNK - 1)
            def _():
                start_gather(c + 1, 1 - j)

            wait_gather(j)

            @pl.when(c >= 2)
            def _():
                wait_store(j)

            compute(c, j)
            start_store(c, j)
        return 0

    lax.fori_loop(0, NCHUNK // 2, step, 0)
    wait_store(0)
    wait_store(1)


@jax.jit
def _run(table, x_resh, a, b):
    mesh = plsc.VectorSubcoreMesh(core_axis_name="c", subcore_axis_name="s")
    kern = pl.kernel(
        _sc_body,
        out_type=jax.ShapeDtypeStruct((ROWS, D), jnp.float32),
        mesh=mesh,
        scratch_types=[
            pltpu.VMEM((NCHUNK, CHUNK), jnp.int32),
            pltpu.VMEM((CHUNK, D), jnp.float32),
            pltpu.VMEM((CHUNK, D), jnp.float32),
            pltpu.VMEM((CHUNK, D), jnp.float32),
            pltpu.VMEM((CHUNK, D), jnp.float32),
            pltpu.VMEM((SEQ, D), jnp.float32),
            pltpu.VMEM((SEQ, D), jnp.float32),
            pltpu.SemaphoreType.DMA,
            pltpu.SemaphoreType.DMA,
            pltpu.SemaphoreType.DMA,
            pltpu.SemaphoreType.DMA,
        ],
        compiler_params=pltpu.CompilerParams(use_tc_tiling_on_sc=False),
        name="rope_embed_sc",
    )
    return kern(table, x_resh, a, b)


def kernel(x, table):
    a, b = _rope_coeffs()
    x_resh = x.astype(jnp.int32).reshape(NW, NCHUNK, CHUNK)
    out = _run(table, x_resh, a, b)
    return out.reshape(BATCH, SEQ, D)


# speedup vs baseline: 5.2446x; 1.2881x over previous
import functools
import math

import jax
import jax.numpy as jnp
from jax import lax
from jax.experimental import pallas as pl
from jax.experimental.pallas import tpu as pltpu
from jax.experimental.pallas import tpu_sc as plsc

VOCAB = 100000
D = 64
BATCH = 1024
SEQ = 200
THETA = 10000.0

NC, NS, L = 2, 16, 16
NW = NC * NS
ROWS = BATCH * SEQ
RPW = ROWS // NW
CHUNK = 128
NCHUNK = RPW // CHUNK


def _rope_coeffs():
    scale = math.sqrt(D)
    freqs = 1.0 / (THETA ** (jnp.arange(0, D, 2).astype(jnp.float32) / D))
    pos = jnp.arange(SEQ).astype(jnp.float32)[:, None]
    cos = jnp.cos(pos * freqs[None, :]) * scale
    sin = jnp.sin(pos * freqs[None, :]) * scale
    a = jnp.repeat(cos, 2, axis=1)
    b = jnp.stack([-sin, sin], axis=-1).reshape(SEQ, D)
    return a, b


_GATHER_DNUMS = lax.GatherDimensionNumbers(
    offset_dims=(), collapsed_slice_dims=(0,), start_index_map=(0,))


def _swap_pairs(v, perm2d):
    return lax.gather(v, perm2d, dimension_numbers=_GATHER_DNUMS,
                      slice_sizes=(1,),
                      mode=lax.GatherScatterMode.PROMISE_IN_BOUNDS)


def _sc_body(table_hbm, x_hbm, a_hbm, b_hbm, out_hbm,
             idx_v, in0, in1, out0, out1, a_v, b_v,
             gsem0, gsem1, ssem0, ssem1):
    w = lax.axis_index("s") * NC + lax.axis_index("c")
    row_base = w * RPW

    pltpu.sync_copy(x_hbm.at[pl.ds(row_base, RPW)], idx_v)
    pltpu.sync_copy(a_hbm, a_v)
    pltpu.sync_copy(b_hbm, b_v)

    perm2d = (lax.iota(jnp.int32, 16) ^ 1)[:, None]

    inbufs = (in0, in1)
    outbufs = (out0, out1)
    gsems = (gsem0, gsem1)
    ssems = (ssem0, ssem1)

    def start_gather(c, j):
        pltpu.async_copy(table_hbm.at[idx_v.at[pl.ds(c * CHUNK, CHUNK)]],
                         inbufs[j], gsems[j])

    def wait_gather(j):
        pltpu.make_async_copy(table_hbm.at[pl.ds(0, CHUNK)],
                              inbufs[j], gsems[j]).wait()

    def start_store(c, j):
        pltpu.async_copy(outbufs[j],
                         out_hbm.at[pl.ds(row_base + c * CHUNK, CHUNK)],
                         ssems[j])

    def wait_store(j):
        pltpu.make_async_copy(outbufs[j],
                              out_hbm.at[pl.ds(row_base, CHUNK)],
                              ssems[j]).wait()

    def compute(c, j):
        pbase = c * CHUNK
        inb = inbufs[j]
        outb = outbufs[j]

        @plsc.parallel_loop(0, CHUNK, 1, unroll=8)
        def row(i):
            p = lax.rem(pbase + i, SEQ)
            for k in range(D // L):
                sl = pl.ds(k * L, L)
                v = inb[i, sl]
                sw = _swap_pairs(v, perm2d)
                outb[i, sl] = v * a_v[p, sl] + sw * b_v[p, sl]

    start_gather(0, 0)

    def step(t, _):
        for j in range(2):
            c = 2 * t + j

            @pl.when(c < NCHUNK - 1)
            def _():
                start_gather(c + 1, 1 - j)

            wait_gather(j)

            @pl.when(c >= 2)
            def _():
                wait_store(j)

            compute(c, j)
            start_store(c, j)
        return 0

    lax.fori_loop(0, NCHUNK // 2, step, 0)
    wait_store(0)
    wait_store(1)


@jax.jit
def _run(table, x_resh, a, b):
    mesh = plsc.VectorSubcoreMesh(core_axis_name="c", subcore_axis_name="s")
    kern = pl.kernel(
        _sc_body,
        out_type=jax.ShapeDtypeStruct((ROWS, D), jnp.float32),
        mesh=mesh,
        scratch_types=[
            pltpu.VMEM((RPW,), jnp.int32),
            pltpu.VMEM((CHUNK, 2 * D), jnp.float32),
            pltpu.VMEM((CHUNK, 2 * D), jnp.float32),
            pltpu.VMEM((CHUNK, D), jnp.float32),
            pltpu.VMEM((CHUNK, D), jnp.float32),
            pltpu.VMEM((SEQ, D), jnp.float32),
            pltpu.VMEM((SEQ, D), jnp.float32),
            pltpu.SemaphoreType.DMA,
            pltpu.SemaphoreType.DMA,
            pltpu.SemaphoreType.DMA,
            pltpu.SemaphoreType.DMA,
        ],
        name="rope_embed_sc",
    )
    return kern(table, x_resh, a, b)


def kernel(x, table):
    a, b = _rope_coeffs()
    x_flat = x.astype(jnp.int32).reshape(ROWS)
    table_pad = jnp.pad(table, ((0, 0), (0, D)))
    out = _run(table_pad, x_flat, a, b)
    return out.reshape(BATCH, SEQ, D)


# speedup vs baseline: 6.3914x; 1.2187x over previous
import math

import jax
import jax.numpy as jnp
from jax import lax
from jax.experimental import pallas as pl
from jax.experimental import layout as jl
from jax.experimental.pallas import tpu as pltpu
from jax.experimental.pallas import tpu_sc as plsc

VOCAB = 100000
D = 64
BATCH = 1024
SEQ = 200
THETA = 10000.0

NC, NS, L = 2, 16, 16
NW = NC * NS
ROWS = BATCH * SEQ
SPW = BATCH // NW
CH = (128, 72)


def _rope_coeffs():
    scale = math.sqrt(D)
    freqs = 1.0 / (THETA ** (jnp.arange(0, D, 2).astype(jnp.float32) / D))
    pos = jnp.arange(SEQ).astype(jnp.float32)[:, None]
    cos = jnp.cos(pos * freqs[None, :]) * scale
    sin = jnp.sin(pos * freqs[None, :]) * scale
    a = jnp.repeat(cos, 2, axis=1)
    b = jnp.stack([-sin, sin], axis=-1).reshape(SEQ, D)
    return a, b


_GATHER_DNUMS = lax.GatherDimensionNumbers(
    offset_dims=(), collapsed_slice_dims=(0,), start_index_map=(0,))


def _swap_pairs(v, perm2d):
    return lax.gather(v, perm2d, dimension_numbers=_GATHER_DNUMS,
                      slice_sizes=(1,),
                      mode=lax.GatherScatterMode.PROMISE_IN_BOUNDS)


def _sc_body(table_hbm, x_hbm, a_hbm, b_hbm, out_hbm,
             idx_v, in0, in1, out0, out1, a_v, b_v,
             gsem0, gsem1, ssem0, ssem1):
    w = lax.axis_index("s") * NC + lax.axis_index("c")
    seq_base = w * SPW

    pltpu.sync_copy(x_hbm.at[pl.ds(seq_base, SPW)], idx_v)
    pltpu.sync_copy(a_hbm, a_v)
    pltpu.sync_copy(b_hbm, b_v)

    perm2d = (lax.iota(jnp.int32, 16) ^ 1)[:, None]

    inbufs = (in0, in1)
    outbufs = (out0, out1)
    gsems = (gsem0, gsem1)
    ssems = (ssem0, ssem1)

    def start_gather(s, j):
        pltpu.async_copy(
            table_hbm.at[idx_v.at[s, pl.ds(j * CH[0], CH[j])]],
            inbufs[j], gsems[j])

    def wait_gather(j):
        pltpu.make_async_copy(table_hbm.at[pl.ds(0, CH[j])],
                              inbufs[j], gsems[j]).wait()

    def start_store(s, j):
        row = (seq_base + s) * SEQ + j * CH[0]
        pltpu.async_copy(outbufs[j], out_hbm.at[pl.ds(row, CH[j])],
                         ssems[j])

    def wait_store(j):
        pltpu.make_async_copy(outbufs[j], out_hbm.at[pl.ds(0, CH[j])],
                              ssems[j]).wait()

    def compute(j):
        pbase = j * CH[0]
        inb = inbufs[j]
        outb = outbufs[j]

        @plsc.parallel_loop(0, CH[j], 1, unroll=8)
        def row(i):
            p = pbase + i
            for k in range(D // L):
                sl = pl.ds(k * L, L)
                v = inb[i, sl]
                sw = _swap_pairs(v, perm2d)
                outb[i, sl] = v * a_v[p, sl] + sw * b_v[p, sl]

    start_gather(0, 0)

    def step(s, _):
        start_gather(s, 1)
        wait_gather(0)

        @pl.when(s >= 1)
        def _():
            wait_store(0)
        compute(0)
        start_store(s, 0)

        @pl.when(s < SPW - 1)
        def _():
            start_gather(s + 1, 0)
        wait_gather(1)

        @pl.when(s >= 1)
        def _():
            wait_store(1)
        compute(1)
        start_store(s, 1)
        return 0

    lax.fori_loop(0, SPW, step, 0)
    wait_store(0)
    wait_store(1)


@jax.jit
def _run(table, x, a, b):
    mesh = plsc.VectorSubcoreMesh(core_axis_name="c", subcore_axis_name="s")
    kern = pl.kernel(
        _sc_body,
        out_type=jax.ShapeDtypeStruct((ROWS, D), jnp.float32),
        mesh=mesh,
        scratch_types=[
            pltpu.VMEM((SPW, SEQ), jnp.int32),
            pltpu.VMEM((CH[0], 2 * D), jnp.float32),
            pltpu.VMEM((CH[1], 2 * D), jnp.float32),
            pltpu.VMEM((CH[0], D), jnp.float32),
            pltpu.VMEM((CH[1], D), jnp.float32),
            pltpu.VMEM((SEQ, D), jnp.float32),
            pltpu.VMEM((SEQ, D), jnp.float32),
            pltpu.SemaphoreType.DMA,
            pltpu.SemaphoreType.DMA,
            pltpu.SemaphoreType.DMA,
            pltpu.SemaphoreType.DMA,
        ],
        name="rope_embed_sc",
    )
    return kern(table, x, a, b)


def kernel(x, table):
    a, b = _rope_coeffs()
    x32 = x.astype(jnp.int32)
    table_pad = jnp.pad(table, ((0, 0), (0, D)))
    out = _run(table_pad, x32, a, b).reshape(BATCH, SEQ, D)
    return jl.with_layout_constraint(out, jl.Layout(major_to_minor=(0, 1, 2)))
